# baseline (speedup 1.0000x reference)
import functools

import jax
import jax.numpy as jnp
from jax import lax
from jax.experimental import pallas as pl
from jax.experimental.pallas import tpu as pltpu
from jax.experimental.pallas import tpu_sc as plsc

N = 4096
D = 768
E = 8
F = 768
TB = 512
T = 128
NB = N // T + E
NPAD = NB * T
NBPAD = 48
NC = 2
NS = 16
NW = NC * NS
CHUNK = N // NW


def _gating_block(x_ref, wg_ref, bg_ref, noise_ref, wt_ref, eid_ref, cnt_ref):
    x = x_ref[...]
    s = lax.dot_general(x, wg_ref[...], (((1,), (1,)), ((), ())),
                        preferred_element_type=jnp.float32)
    s = s + bg_ref[...] + noise_ref[...]
    m = jnp.max(s, axis=1, keepdims=True)
    p = jnp.exp(s - m)
    gw = p / jnp.sum(p, axis=1, keepdims=True)
    wt = jnp.max(gw, axis=1, keepdims=True)
    ii = lax.broadcasted_iota(jnp.int32, (TB, E), 1)
    eid = jnp.min(jnp.where(gw == wt, ii, E), axis=1, keepdims=True)
    wt_ref[...] = wt
    eid_ref[...] = eid
    oh = (eid == lax.broadcasted_iota(jnp.int32, (TB, 16), 1)).astype(jnp.int32)
    for c in range(TB // CHUNK):
        cnt_ref[0, pl.ds(c, 1), :] = jnp.sum(
            oh[c * CHUNK:(c + 1) * CHUNK], axis=0, keepdims=True)


def _gating(x_flat, Wg, bg, noise):
    return pl.pallas_call(
        _gating_block,
        grid=(N // TB,),
        in_specs=[
            pl.BlockSpec((TB, D), lambda i: (i, 0)),
            pl.BlockSpec((E, D), lambda i: (0, 0)),
            pl.BlockSpec((1, E), lambda i: (0, 0)),
            pl.BlockSpec((TB, E), lambda i: (i, 0)),
        ],
        out_specs=[
            pl.BlockSpec((TB, 1), lambda i: (i, 0)),
            pl.BlockSpec((TB, 1), lambda i: (i, 0)),
            pl.BlockSpec((1, TB // CHUNK, 16), lambda i: (i, 0, 0)),
        ],
        out_shape=[
            jax.ShapeDtypeStruct((N, 1), jnp.float32),
            jax.ShapeDtypeStruct((N, 1), jnp.int32),
            jax.ShapeDtypeStruct((N // TB, TB // CHUNK, 16), jnp.int32),
        ],
    )(x_flat, Wg, bg.reshape(1, E), noise)


def _dispatch_body(e_hbm, cnt_hbm, x_hbm, xd_hbm, slots_hbm, be_hbm,
                   ev, cv, slots_v, xbuf, be_vm, sem):
    wid = lax.axis_index("c") * NS + lax.axis_index("s")
    base_tok = wid * CHUNK
    pltpu.sync_copy(e_hbm.at[pl.ds(base_tok, CHUNK)], ev)
    pltpu.sync_copy(cnt_hbm, cv)

    iota = lax.iota(jnp.int32, 16)
    widv = lax.broadcast_in_dim(wid, (16,), ())
    tot = jnp.zeros((16,), jnp.int32)
    pre = jnp.zeros((16,), jnp.int32)
    for r in range(NW):
        v = cv[pl.ds(r * 16, 16)]
        tot = tot + v
        rv = jnp.full((16,), r, jnp.int32)
        pre = pre + jnp.where(rv < widv, v, 0)
    totpad = ((tot + (T - 1)) >> 7) << 7
    ends = plsc.cumsum(totpad)
    base = (ends - totpad) + pre

    for j in range(CHUNK // 16):
        v = ev[pl.ds(j * 16, 16)]
        sj = jnp.zeros((16,), jnp.int32)
        for ex in range(E):
            msk = v == ex
            rk = plsc.cumsum(msk.astype(jnp.int32)) - 1
            bsc = jnp.sum(jnp.where(iota == ex, base, 0))
            sj = jnp.where(msk, bsc + rk, sj)
            csp = plsc.all_reduce_population_count(msk)
            base = base + jnp.where(iota == ex, csp, 0)
        slots_v[pl.ds(j * 16, 16)] = sj
    pltpu.sync_copy(slots_v, slots_hbm.at[pl.ds(base_tok, CHUNK)])

    pltpu.sync_copy(x_hbm.at[pl.ds(base_tok, CHUNK)], xbuf)
    pltpu.async_copy(xbuf, xd_hbm.at[slots_v], sem).wait()

    for g in range(NBPAD // 16):
        ids = iota + g * 16
        it = ids * T
        cnt = jnp.zeros((16,), jnp.int32)
        for ex in range(E):
            esc = jnp.sum(jnp.where(iota == ex, ends, 0))
            cnt = cnt + jnp.where(esc <= it, 1, 0)
        be_vm[pl.ds(g * 16, 16)] = jnp.minimum(cnt, E - 1)

    @pl.when(wid == 0)
    def _():
        pltpu.sync_copy(be_vm, be_hbm)


def _dispatch(e1d, counts_flat, x_flat):
    mesh = plsc.VectorSubcoreMesh(core_axis_name="c", subcore_axis_name="s",
                                  num_cores=NC, num_subcores=NS)
    f = pl.kernel(
        _dispatch_body,
        out_type=[
            jax.ShapeDtypeStruct((NPAD, D), jnp.float32),
            jax.ShapeDtypeStruct((N,), jnp.int32),
            jax.ShapeDtypeStruct((NBPAD,), jnp.int32),
        ],
        mesh=mesh,
        scratch_types=[
            pltpu.VMEM((CHUNK,), jnp.int32),
            pltpu.VMEM((NW * 16,), jnp.int32),
            pltpu.VMEM((CHUNK,), jnp.int32),
            pltpu.VMEM((CHUNK, D), jnp.float32),
            pltpu.VMEM((NBPAD,), jnp.int32),
            pltpu.SemaphoreType.DMA,
        ],
        compiler_params=pltpu.CompilerParams(needs_layout_passes=False),
    )
    return f(e1d, counts_flat, x_flat)


def _expert_block(be_ref, x_ref, we_ref, bex_ref, h_ref):
    h = lax.dot_general(x_ref[...], we_ref[0], (((1,), (1,)), ((), ())),
                        preferred_element_type=jnp.float32)
    h_ref[...] = h + bex_ref[0]


def _expert_mm(be_arr, x_disp, W_experts, b_experts):
    grid_spec = pltpu.PrefetchScalarGridSpec(
        num_scalar_prefetch=1,
        grid=(NB,),
        in_specs=[
            pl.BlockSpec((T, D), lambda i, be: (i, 0)),
            pl.BlockSpec((1, F, D), lambda i, be: (be[i], 0, 0)),
            pl.BlockSpec((1, 1, F), lambda i, be: (be[i], 0, 0)),
        ],
        out_specs=pl.BlockSpec((T, F), lambda i, be: (i, 0)),
    )
    return pl.pallas_call(
        _expert_block,
        grid_spec=grid_spec,
        out_shape=jax.ShapeDtypeStruct((NPAD, F), jnp.float32),
    )(be_arr, x_disp, W_experts, b_experts.reshape(E, 1, F))


def _gather_body(slots_hbm, h_hbm, out_hbm, sv, hbuf, sem):
    wid = lax.axis_index("c") * NS + lax.axis_index("s")
    base_tok = wid * CHUNK
    pltpu.sync_copy(slots_hbm.at[pl.ds(base_tok, CHUNK)], sv)
    pltpu.async_copy(h_hbm.at[sv], hbuf, sem).wait()
    pltpu.sync_copy(hbuf, out_hbm.at[pl.ds(base_tok, CHUNK)])


def _gather(slots, h_disp):
    mesh = plsc.VectorSubcoreMesh(core_axis_name="c", subcore_axis_name="s",
                                  num_cores=NC, num_subcores=NS)
    f = pl.kernel(
        _gather_body,
        out_type=jax.ShapeDtypeStruct((N, F), jnp.float32),
        mesh=mesh,
        scratch_types=[
            pltpu.VMEM((CHUNK,), jnp.int32),
            pltpu.VMEM((CHUNK, F), jnp.float32),
            pltpu.SemaphoreType.DMA,
        ],
        compiler_params=pltpu.CompilerParams(needs_layout_passes=False),
    )
    return f(slots, h_disp)


def _proj_block(h_ref, wp_ref, bp_ref, wt_ref, out_ref):
    y = lax.dot_general(h_ref[...], wp_ref[...], (((1,), (1,)), ((), ())),
                        preferred_element_type=jnp.float32)
    out_ref[...] = wt_ref[...] * (y + bp_ref[...])


def _projection(h_tok, Wp, bp, wt):
    return pl.pallas_call(
        _proj_block,
        grid=(N // TB,),
        in_specs=[
            pl.BlockSpec((TB, F), lambda i: (i, 0)),
            pl.BlockSpec((D, F), lambda i: (0, 0)),
            pl.BlockSpec((1, D), lambda i: (0, 0)),
            pl.BlockSpec((TB, 1), lambda i: (i, 0)),
        ],
        out_specs=pl.BlockSpec((TB, D), lambda i: (i, 0)),
        out_shape=jax.ShapeDtypeStruct((N, D), jnp.float32),
    )(h_tok, Wp, bp.reshape(1, D), wt)


def kernel(x, Wg, bg, W_experts, b_experts, Wp, bp, noise):
    orig_shape = x.shape
    x_flat = x.reshape(N, D)
    wt, eid, counts = _gating(x_flat, Wg, bg, noise)
    x_disp, slots, be_arr = _dispatch(eid.reshape(N), counts.reshape(NW * 16),
                                      x_flat)
    h_disp = _expert_mm(be_arr, x_disp, W_experts, b_experts)
    h_tok = _gather(slots, h_disp)
    out = _projection(h_tok, Wp, bp, wt)
    return out.reshape(orig_shape)


# speedup vs baseline: 1.0065x; 1.0065x over previous
import jax
import jax.numpy as jnp
from jax import lax
from jax.experimental import pallas as pl
from jax.experimental.pallas import tpu as pltpu
from jax.experimental.pallas import tpu_sc as plsc

N = 4096
D = 768
E = 8
F = 768
TB = 512
T = 128
NB = N // T + E
NPAD = NB * T
NBPAD = 48
NC = 2
NS = 16
NW = NC * NS
CHUNK = N // NW


def _gating_block(x_ref, wg_ref, bg_ref, noise_ref, wt_ref, eid_ref, cnt_ref):
    x = x_ref[...]
    s = lax.dot_general(x, wg_ref[...], (((1,), (1,)), ((), ())),
                        preferred_element_type=jnp.float32)
    s = s + bg_ref[...] + noise_ref[...]
    m = jnp.max(s, axis=1, keepdims=True)
    p = jnp.exp(s - m)
    gw = p / jnp.sum(p, axis=1, keepdims=True)
    wt = jnp.max(gw, axis=1, keepdims=True)
    ii = lax.broadcasted_iota(jnp.int32, (TB, E), 1)
    eid = jnp.min(jnp.where(gw == wt, ii, E), axis=1, keepdims=True)
    wt_ref[...] = wt
    eid_ref[...] = eid
    oh = (eid == lax.broadcasted_iota(jnp.int32, (TB, 16), 1)).astype(jnp.int32)
    for c in range(TB // CHUNK):
        cnt_ref[0, pl.ds(c, 1), :] = jnp.sum(
            oh[c * CHUNK:(c + 1) * CHUNK], axis=0, keepdims=True)


def _gating(x_flat, Wg, bg, noise):
    return pl.pallas_call(
        _gating_block,
        grid=(N // TB,),
        in_specs=[
            pl.BlockSpec((TB, D), lambda i: (i, 0)),
            pl.BlockSpec((E, D), lambda i: (0, 0)),
            pl.BlockSpec((1, E), lambda i: (0, 0)),
            pl.BlockSpec((TB, E), lambda i: (i, 0)),
        ],
        out_specs=[
            pl.BlockSpec((TB, 1), lambda i: (i, 0)),
            pl.BlockSpec((TB, 1), lambda i: (i, 0)),
            pl.BlockSpec((1, TB // CHUNK, 16), lambda i: (i, 0, 0)),
        ],
        out_shape=[
            jax.ShapeDtypeStruct((N, 1), jnp.float32),
            jax.ShapeDtypeStruct((N, 1), jnp.int32),
            jax.ShapeDtypeStruct((N // TB, TB // CHUNK, 16), jnp.int32),
        ],
    )(x_flat, Wg, bg.reshape(1, E), noise)


def _dispatch_body(e_hbm, cnt_hbm, x_hbm, w_hbm, xd_hbm, wd_hbm, slots_hbm,
                   be_hbm, ev, cv, wv, wbuf, slots_v, xbuf, be_vm, sem):
    wid = lax.axis_index("c") * NS + lax.axis_index("s")
    base_tok = wid * CHUNK
    pltpu.sync_copy(e_hbm.at[pl.ds(base_tok, CHUNK)], ev)
    pltpu.sync_copy(w_hbm.at[pl.ds(base_tok, CHUNK)], wv)
    pltpu.sync_copy(cnt_hbm, cv)

    iota = lax.iota(jnp.int32, 16)
    widv = lax.broadcast_in_dim(wid, (16,), ())
    tot = jnp.zeros((16,), jnp.int32)
    pre = jnp.zeros((16,), jnp.int32)
    for r in range(NW):
        v = cv[pl.ds(r * 16, 16)]
        tot = tot + v
        rv = jnp.full((16,), r, jnp.int32)
        pre = pre + jnp.where(rv < widv, v, 0)
    totpad = ((tot + (T - 1)) >> 7) << 7
    ends = plsc.cumsum(totpad)
    base = (ends - totpad) + pre

    zero16 = jnp.zeros((16,), jnp.int32)
    for j in range(CHUNK // 16):
        v = ev[pl.ds(j * 16, 16)]
        sj = jnp.zeros((16,), jnp.int32)
        for ex in range(E):
            msk = v == ex
            rk = plsc.cumsum(msk.astype(jnp.int32)) - 1
            bsc = jnp.sum(jnp.where(iota == ex, base, 0))
            sj = jnp.where(msk, bsc + rk, sj)
            csp = plsc.all_reduce_population_count(msk)
            base = base + jnp.where(iota == ex, csp, 0)
        slots_v[pl.ds(j * 16, 16)] = sj
        plsc.store_scatter(wbuf, [iota + j * 16, zero16],
                           wv[pl.ds(j * 16, 16)])
    pltpu.sync_copy(slots_v, slots_hbm.at[pl.ds(base_tok, CHUNK)])

    pltpu.sync_copy(x_hbm.at[pl.ds(base_tok, CHUNK)], xbuf)
    cp_x = pltpu.async_copy(xbuf, xd_hbm.at[slots_v], sem)
    cp_w = pltpu.async_copy(wbuf, wd_hbm.at[slots_v], sem)

    for g in range(NBPAD // 16):
        ids = iota + g * 16
        it = ids * T
        cnt = jnp.zeros((16,), jnp.int32)
        for ex in range(E):
            esc = jnp.sum(jnp.where(iota == ex, ends, 0))
            cnt = cnt + jnp.where(esc <= it, 1, 0)
        be_vm[pl.ds(g * 16, 16)] = jnp.minimum(cnt, E - 1)

    @pl.when(wid == 0)
    def _():
        pltpu.sync_copy(be_vm, be_hbm)

    cp_x.wait()
    cp_w.wait()


def _dispatch(e1d, counts_flat, x_flat, wt1d):
    mesh = plsc.VectorSubcoreMesh(core_axis_name="c", subcore_axis_name="s",
                                  num_cores=NC, num_subcores=NS)
    f = pl.kernel(
        _dispatch_body,
        out_type=[
            jax.ShapeDtypeStruct((NPAD, D), jnp.float32),
            jax.ShapeDtypeStruct((NPAD, 128), jnp.float32),
            jax.ShapeDtypeStruct((N,), jnp.int32),
            jax.ShapeDtypeStruct((NBPAD,), jnp.int32),
        ],
        mesh=mesh,
        scratch_types=[
            pltpu.VMEM((CHUNK,), jnp.int32),
            pltpu.VMEM((NW * 16,), jnp.int32),
            pltpu.VMEM((CHUNK,), jnp.float32),
            pltpu.VMEM((CHUNK, 128), jnp.float32),
            pltpu.VMEM((CHUNK,), jnp.int32),
            pltpu.VMEM((CHUNK, D), jnp.float32),
            pltpu.VMEM((NBPAD,), jnp.int32),
            pltpu.SemaphoreType.DMA,
        ],
        compiler_params=pltpu.CompilerParams(needs_layout_passes=False),
    )
    return f(e1d, counts_flat, x_flat, wt1d)


def _expert_block(be_ref, x_ref, we_ref, bex_ref, wp_ref, bp_ref, wd_ref,
                  y_ref):
    i = pl.program_id(0)
    ex = be_ref[i]
    h = lax.dot_general(x_ref[...], we_ref[ex], (((1,), (1,)), ((), ())),
                        preferred_element_type=jnp.float32)
    h = h + bex_ref[0]
    y = lax.dot_general(h, wp_ref[...], (((1,), (1,)), ((), ())),
                        preferred_element_type=jnp.float32)
    y_ref[...] = (y + bp_ref[...]) * wd_ref[:, 0:1]


def _expert_mm(be_arr, x_disp, w_disp, W_experts, b_experts, Wp, bp):
    grid_spec = pltpu.PrefetchScalarGridSpec(
        num_scalar_prefetch=1,
        grid=(NB,),
        in_specs=[
            pl.BlockSpec((T, D), lambda i, be: (i, 0)),
            pl.BlockSpec((E, F, D), lambda i, be: (0, 0, 0)),
            pl.BlockSpec((1, 1, F), lambda i, be: (be[i], 0, 0)),
            pl.BlockSpec((D, F), lambda i, be: (0, 0)),
            pl.BlockSpec((1, D), lambda i, be: (0, 0)),
            pl.BlockSpec((T, 128), lambda i, be: (i, 0)),
        ],
        out_specs=pl.BlockSpec((T, D), lambda i, be: (i, 0)),
    )
    return pl.pallas_call(
        _expert_block,
        grid_spec=grid_spec,
        out_shape=jax.ShapeDtypeStruct((NPAD, D), jnp.float32),
    )(be_arr, x_disp, W_experts, b_experts.reshape(E, 1, F), Wp,
      bp.reshape(1, D), w_disp)


def _gather_body(slots_hbm, y_hbm, out_hbm, sv, ybuf, sem):
    wid = lax.axis_index("c") * NS + lax.axis_index("s")
    base_tok = wid * CHUNK
    pltpu.sync_copy(slots_hbm.at[pl.ds(base_tok, CHUNK)], sv)
    pltpu.async_copy(y_hbm.at[sv], ybuf, sem).wait()
    pltpu.sync_copy(ybuf, out_hbm.at[pl.ds(base_tok, CHUNK)])


def _gather(slots, y_disp):
    mesh = plsc.VectorSubcoreMesh(core_axis_name="c", subcore_axis_name="s",
                                  num_cores=NC, num_subcores=NS)
    f = pl.kernel(
        _gather_body,
        out_type=jax.ShapeDtypeStruct((N, D), jnp.float32),
        mesh=mesh,
        scratch_types=[
            pltpu.VMEM((CHUNK,), jnp.int32),
            pltpu.VMEM((CHUNK, D), jnp.float32),
            pltpu.SemaphoreType.DMA,
        ],
        compiler_params=pltpu.CompilerParams(needs_layout_passes=False),
    )
    return f(slots, y_disp)


def kernel(x, Wg, bg, W_experts, b_experts, Wp, bp, noise):
    orig_shape = x.shape
    x_flat = x.reshape(N, D)
    wt, eid, counts = _gating(x_flat, Wg, bg, noise)
    x_disp, w_disp, slots, be_arr = _dispatch(
        eid.reshape(N), counts.reshape(NW * 16), x_flat, wt.reshape(N))
    y_disp = _expert_mm(be_arr, x_disp, w_disp, W_experts, b_experts, Wp, bp)
    out = _gather(slots, y_disp)
    return out.reshape(orig_shape)


# speedup vs baseline: 1.1079x; 1.1007x over previous
import jax
import jax.numpy as jnp
from jax import lax
from jax.experimental import pallas as pl
from jax.experimental.pallas import tpu as pltpu
from jax.experimental.pallas import tpu_sc as plsc

N = 4096
D = 768
E = 8
F = 768
TB = 512
T = 256
LOG2T = 8
NB = N // T + E
NPAD = NB * T
NBPAD = 32
NC = 2
NS = 16
NW = NC * NS
CHUNK = N // NW


def _gating_block(x_ref, wg_ref, bg_ref, noise_ref, wt_ref, eid_ref, cnt_ref):
    x = x_ref[...]
    s = lax.dot_general(x, wg_ref[...], (((1,), (1,)), ((), ())),
                        preferred_element_type=jnp.float32)
    s = s + bg_ref[...] + noise_ref[...]
    m = jnp.max(s, axis=1, keepdims=True)
    p = jnp.exp(s - m)
    gw = p / jnp.sum(p, axis=1, keepdims=True)
    wt = jnp.max(gw, axis=1, keepdims=True)
    ii = lax.broadcasted_iota(jnp.int32, (TB, E), 1)
    eid = jnp.min(jnp.where(gw == wt, ii, E), axis=1, keepdims=True)
    wt_ref[...] = wt
    eid_ref[...] = eid
    oh = (eid == lax.broadcasted_iota(jnp.int32, (TB, 16), 1)).astype(jnp.int32)
    for c in range(TB // CHUNK):
        cnt_ref[0, pl.ds(c, 1), :] = jnp.sum(
            oh[c * CHUNK:(c + 1) * CHUNK], axis=0, keepdims=True)


def _gating(x_flat, Wg, bg, noise):
    return pl.pallas_call(
        _gating_block,
        grid=(N // TB,),
        in_specs=[
            pl.BlockSpec((TB, D), lambda i: (i, 0)),
            pl.BlockSpec((E, D), lambda i: (0, 0)),
            pl.BlockSpec((1, E), lambda i: (0, 0)),
            pl.BlockSpec((TB, E), lambda i: (i, 0)),
        ],
        out_specs=[
            pl.BlockSpec((TB, 1), lambda i: (i, 0)),
            pl.BlockSpec((TB, 1), lambda i: (i, 0)),
            pl.BlockSpec((1, TB // CHUNK, 16), lambda i: (i, 0, 0)),
        ],
        out_shape=[
            jax.ShapeDtypeStruct((N, 1), jnp.float32),
            jax.ShapeDtypeStruct((N, 1), jnp.int32),
            jax.ShapeDtypeStruct((N // TB, TB // CHUNK, 16), jnp.int32),
        ],
    )(x_flat, Wg, bg.reshape(1, E), noise)


def _dispatch_body(e_hbm, cnt_hbm, x_hbm, w_hbm, xd_hbm, wd_hbm, slots_hbm,
                   be_hbm, ev, cv, wv, wbuf, slots_v, xbuf, be_vm, sem):
    wid = lax.axis_index("c") * NS + lax.axis_index("s")
    base_tok = wid * CHUNK
    pltpu.sync_copy(e_hbm.at[pl.ds(base_tok, CHUNK)], ev)
    pltpu.sync_copy(w_hbm.at[pl.ds(base_tok, CHUNK)], wv)
    pltpu.sync_copy(cnt_hbm, cv)

    iota = lax.iota(jnp.int32, 16)
    widv = lax.broadcast_in_dim(wid, (16,), ())
    tot = jnp.zeros((16,), jnp.int32)
    pre = jnp.zeros((16,), jnp.int32)
    for r in range(NW):
        v = cv[pl.ds(r * 16, 16)]
        tot = tot + v
        rv = jnp.full((16,), r, jnp.int32)
        pre = pre + jnp.where(rv < widv, v, 0)
    totpad = ((tot + (T - 1)) >> LOG2T) << LOG2T
    ends = plsc.cumsum(totpad)
    base = (ends - totpad) + pre

    zero16 = jnp.zeros((16,), jnp.int32)
    for j in range(CHUNK // 16):
        v = ev[pl.ds(j * 16, 16)]
        sj = jnp.zeros((16,), jnp.int32)
        for ex in range(E):
            msk = v == ex
            rk = plsc.cumsum(msk.astype(jnp.int32)) - 1
            bsc = jnp.sum(jnp.where(iota == ex, base, 0))
            sj = jnp.where(msk, bsc + rk, sj)
            csp = plsc.all_reduce_population_count(msk)
            base = base + jnp.where(iota == ex, csp, 0)
        slots_v[pl.ds(j * 16, 16)] = sj
        plsc.store_scatter(wbuf, [iota + j * 16, zero16],
                           wv[pl.ds(j * 16, 16)])
    pltpu.sync_copy(slots_v, slots_hbm.at[pl.ds(base_tok, CHUNK)])

    pltpu.sync_copy(x_hbm.at[pl.ds(base_tok, CHUNK)], xbuf)
    cp_x = pltpu.async_copy(xbuf, xd_hbm.at[slots_v], sem)
    cp_w = pltpu.async_copy(wbuf, wd_hbm.at[slots_v], sem)

    for g in range(NBPAD // 16):
        ids = iota + g * 16
        it = ids * T
        cnt = jnp.zeros((16,), jnp.int32)
        for ex in range(E):
            esc = jnp.sum(jnp.where(iota == ex, ends, 0))
            cnt = cnt + jnp.where(esc <= it, 1, 0)
        be_vm[pl.ds(g * 16, 16)] = jnp.minimum(cnt, E - 1)

    @pl.when(wid == 0)
    def _():
        pltpu.sync_copy(be_vm, be_hbm)

    cp_x.wait()
    cp_w.wait()


def _dispatch(e1d, counts_flat, x_flat, wt1d):
    mesh = plsc.VectorSubcoreMesh(core_axis_name="c", subcore_axis_name="s",
                                  num_cores=NC, num_subcores=NS)
    f = pl.kernel(
        _dispatch_body,
        out_type=[
            jax.ShapeDtypeStruct((NPAD, D), jnp.float32),
            jax.ShapeDtypeStruct((NPAD, 128), jnp.float32),
            jax.ShapeDtypeStruct((N,), jnp.int32),
            jax.ShapeDtypeStruct((NBPAD,), jnp.int32),
        ],
        mesh=mesh,
        scratch_types=[
            pltpu.VMEM((CHUNK,), jnp.int32),
            pltpu.VMEM((NW * 16,), jnp.int32),
            pltpu.VMEM((CHUNK,), jnp.float32),
            pltpu.VMEM((CHUNK, 128), jnp.float32),
            pltpu.VMEM((CHUNK,), jnp.int32),
            pltpu.VMEM((CHUNK, D), jnp.float32),
            pltpu.VMEM((NBPAD,), jnp.int32),
            pltpu.SemaphoreType.DMA,
        ],
        compiler_params=pltpu.CompilerParams(needs_layout_passes=False),
    )
    return f(e1d, counts_flat, x_flat, wt1d)


def _expert_block(be_ref, x_ref, we_ref, bex_ref, wp_ref, bp_ref, wd_ref,
                  y_ref):
    xb = x_ref[...].astype(jnp.bfloat16)
    h = lax.dot_general(xb, we_ref[0], (((1,), (1,)), ((), ())),
                        preferred_element_type=jnp.float32)
    h = h + bex_ref[0]
    y = lax.dot_general(h.astype(jnp.bfloat16), wp_ref[...], (((1,), (1,)), ((), ())),
                        preferred_element_type=jnp.float32)
    y_ref[...] = (y + bp_ref[...]) * wd_ref[:, 0:1]


def _expert_mm(be_arr, x_disp, w_disp, W_experts, b_experts, Wp, bp):
    grid_spec = pltpu.PrefetchScalarGridSpec(
        num_scalar_prefetch=1,
        grid=(NB,),
        in_specs=[
            pl.BlockSpec((T, D), lambda i, be: (i, 0)),
            pl.BlockSpec((1, F, D), lambda i, be: (be[i], 0, 0)),
            pl.BlockSpec((1, 1, F), lambda i, be: (be[i], 0, 0)),
            pl.BlockSpec((D, F), lambda i, be: (0, 0)),
            pl.BlockSpec((1, D), lambda i, be: (0, 0)),
            pl.BlockSpec((T, 128), lambda i, be: (i, 0)),
        ],
        out_specs=pl.BlockSpec((T, D), lambda i, be: (i, 0)),
    )
    return pl.pallas_call(
        _expert_block,
        grid_spec=grid_spec,
        out_shape=jax.ShapeDtypeStruct((NPAD, D), jnp.float32),
    )(be_arr, x_disp, W_experts.astype(jnp.bfloat16),
      b_experts.reshape(E, 1, F), Wp.astype(jnp.bfloat16),
      bp.reshape(1, D), w_disp)


def _gather_body(slots_hbm, y_hbm, out_hbm, sv, ybuf, sem):
    wid = lax.axis_index("c") * NS + lax.axis_index("s")
    base_tok = wid * CHUNK
    pltpu.sync_copy(slots_hbm.at[pl.ds(base_tok, CHUNK)], sv)
    pltpu.async_copy(y_hbm.at[sv], ybuf, sem).wait()
    pltpu.sync_copy(ybuf, out_hbm.at[pl.ds(base_tok, CHUNK)])


def _gather(slots, y_disp):
    mesh = plsc.VectorSubcoreMesh(core_axis_name="c", subcore_axis_name="s",
                                  num_cores=NC, num_subcores=NS)
    f = pl.kernel(
        _gather_body,
        out_type=jax.ShapeDtypeStruct((N, D), jnp.float32),
        mesh=mesh,
        scratch_types=[
            pltpu.VMEM((CHUNK,), jnp.int32),
            pltpu.VMEM((CHUNK, D), jnp.float32),
            pltpu.SemaphoreType.DMA,
        ],
        compiler_params=pltpu.CompilerParams(needs_layout_passes=False),
    )
    return f(slots, y_disp)


def kernel(x, Wg, bg, W_experts, b_experts, Wp, bp, noise):
    orig_shape = x.shape
    x_flat = x.reshape(N, D)
    wt, eid, counts = _gating(x_flat, Wg, bg, noise)
    x_disp, w_disp, slots, be_arr = _dispatch(
        eid.reshape(N), counts.reshape(NW * 16), x_flat, wt.reshape(N))
    y_disp = _expert_mm(be_arr, x_disp, w_disp, W_experts, b_experts, Wp, bp)
    out = _gather(slots, y_disp)
    return out.reshape(orig_shape)


# speedup vs baseline: 1.1315x; 1.0213x over previous
import jax
import jax.numpy as jnp
from jax import lax
from jax.experimental import pallas as pl
from jax.experimental.pallas import tpu as pltpu
from jax.experimental.pallas import tpu_sc as plsc

N = 4096
D = 768
E = 8
F = 768
TB = 512
T = 512
LOG2T = 9
NB = N // T + E
NPAD = NB * T
NBPAD = 32
NC = 2
NS = 16
NW = NC * NS
CHUNK = N // NW


def _gating_block(x_ref, wg_ref, bg_ref, noise_ref, wt_ref, eid_ref, cnt_ref):
    x = x_ref[...]
    s = lax.dot_general(x, wg_ref[...], (((1,), (1,)), ((), ())),
                        preferred_element_type=jnp.float32)
    s = s + bg_ref[...] + noise_ref[...]
    m = jnp.max(s, axis=1, keepdims=True)
    p = jnp.exp(s - m)
    gw = p / jnp.sum(p, axis=1, keepdims=True)
    wt = jnp.max(gw, axis=1, keepdims=True)
    ii = lax.broadcasted_iota(jnp.int32, (TB, E), 1)
    eid = jnp.min(jnp.where(gw == wt, ii, E), axis=1, keepdims=True)
    wt_ref[...] = wt
    eid_ref[...] = eid
    oh = (eid == lax.broadcasted_iota(jnp.int32, (TB, 16), 1)).astype(jnp.int32)
    for c in range(TB // CHUNK):
        cnt_ref[0, pl.ds(c, 1), :] = jnp.sum(
            oh[c * CHUNK:(c + 1) * CHUNK], axis=0, keepdims=True)


def _gating(x_flat, Wg, bg, noise):
    return pl.pallas_call(
        _gating_block,
        grid=(N // TB,),
        in_specs=[
            pl.BlockSpec((TB, D), lambda i: (i, 0)),
            pl.BlockSpec((E, D), lambda i: (0, 0)),
            pl.BlockSpec((1, E), lambda i: (0, 0)),
            pl.BlockSpec((TB, E), lambda i: (i, 0)),
        ],
        out_specs=[
            pl.BlockSpec((TB, 1), lambda i: (i, 0)),
            pl.BlockSpec((TB, 1), lambda i: (i, 0)),
            pl.BlockSpec((1, TB // CHUNK, 16), lambda i: (i, 0, 0)),
        ],
        out_shape=[
            jax.ShapeDtypeStruct((N, 1), jnp.float32),
            jax.ShapeDtypeStruct((N, 1), jnp.int32),
            jax.ShapeDtypeStruct((N // TB, TB // CHUNK, 16), jnp.int32),
        ],
    )(x_flat, Wg, bg.reshape(1, E), noise)


def _dispatch_body(e_hbm, cnt_hbm, x_hbm, xd_hbm, slots_hbm,
                   be_hbm, ev, cv, slots_v, xbuf, be_vm, sem):
    wid = lax.axis_index("c") * NS + lax.axis_index("s")
    base_tok = wid * CHUNK
    pltpu.sync_copy(e_hbm.at[pl.ds(base_tok, CHUNK)], ev)
    pltpu.sync_copy(cnt_hbm, cv)

    iota = lax.iota(jnp.int32, 16)
    widv = lax.broadcast_in_dim(wid, (16,), ())
    tot = jnp.zeros((16,), jnp.int32)
    pre = jnp.zeros((16,), jnp.int32)
    for r in range(NW):
        v = cv[pl.ds(r * 16, 16)]
        tot = tot + v
        rv = jnp.full((16,), r, jnp.int32)
        pre = pre + jnp.where(rv < widv, v, 0)
    totpad = ((tot + (T - 1)) >> LOG2T) << LOG2T
    ends = plsc.cumsum(totpad)
    base = (ends - totpad) + pre

    for j in range(CHUNK // 16):
        v = ev[pl.ds(j * 16, 16)]
        sj = jnp.zeros((16,), jnp.int32)
        for ex in range(E):
            msk = v == ex
            rk = plsc.cumsum(msk.astype(jnp.int32)) - 1
            bsc = jnp.sum(jnp.where(iota == ex, base, 0))
            sj = jnp.where(msk, bsc + rk, sj)
            csp = plsc.all_reduce_population_count(msk)
            base = base + jnp.where(iota == ex, csp, 0)
        slots_v[pl.ds(j * 16, 16)] = sj
    pltpu.sync_copy(slots_v, slots_hbm.at[pl.ds(base_tok, CHUNK)])

    pltpu.sync_copy(x_hbm.at[pl.ds(base_tok, CHUNK)], xbuf)
    cp_x = pltpu.async_copy(xbuf, xd_hbm.at[slots_v], sem)

    for g in range(NBPAD // 16):
        ids = iota + g * 16
        it = ids * T
        cnt = jnp.zeros((16,), jnp.int32)
        for ex in range(E):
            esc = jnp.sum(jnp.where(iota == ex, ends, 0))
            cnt = cnt + jnp.where(esc <= it, 1, 0)
        be_vm[pl.ds(g * 16, 16)] = jnp.minimum(cnt, E - 1)

    @pl.when(wid == 0)
    def _():
        pltpu.sync_copy(be_vm, be_hbm)

    cp_x.wait()


def _dispatch(e1d, counts_flat, x_flat):
    mesh = plsc.VectorSubcoreMesh(core_axis_name="c", subcore_axis_name="s",
                                  num_cores=NC, num_subcores=NS)
    f = pl.kernel(
        _dispatch_body,
        out_type=[
            jax.ShapeDtypeStruct((NPAD, D), jnp.float32),
            jax.ShapeDtypeStruct((N,), jnp.int32),
            jax.ShapeDtypeStruct((NBPAD,), jnp.int32),
        ],
        mesh=mesh,
        scratch_types=[
            pltpu.VMEM((CHUNK,), jnp.int32),
            pltpu.VMEM((NW * 16,), jnp.int32),
            pltpu.VMEM((CHUNK,), jnp.int32),
            pltpu.VMEM((CHUNK, D), jnp.float32),
            pltpu.VMEM((NBPAD,), jnp.int32),
            pltpu.SemaphoreType.DMA,
        ],
        compiler_params=pltpu.CompilerParams(needs_layout_passes=False),
    )
    return f(e1d, counts_flat, x_flat)


def _expert_block(be_ref, x_ref, we_ref, bex_ref, h_ref):
    h = lax.dot_general(x_ref[...], we_ref[0], (((1,), (1,)), ((), ())),
                        preferred_element_type=jnp.float32)
    h_ref[...] = h + bex_ref[0]


def _expert_mm(be_arr, x_disp, W_experts, b_experts):
    grid_spec = pltpu.PrefetchScalarGridSpec(
        num_scalar_prefetch=1,
        grid=(NB,),
        in_specs=[
            pl.BlockSpec((T, D), lambda i, be: (i, 0)),
            pl.BlockSpec((1, F, D), lambda i, be: (be[i], 0, 0)),
            pl.BlockSpec((1, 1, F), lambda i, be: (be[i], 0, 0)),
        ],
        out_specs=pl.BlockSpec((T, F), lambda i, be: (i, 0)),
    )
    return pl.pallas_call(
        _expert_block,
        grid_spec=grid_spec,
        out_shape=jax.ShapeDtypeStruct((NPAD, F), jnp.float32),
    )(be_arr, x_disp, W_experts, b_experts.reshape(E, 1, F))


def _proj_block(h_ref, wp_ref, bp_ref, wt_ref, out_ref):
    y = lax.dot_general(h_ref[...], wp_ref[...], (((1,), (1,)), ((), ())),
                        preferred_element_type=jnp.float32)
    out_ref[...] = (y + bp_ref[...]) * wt_ref[...]


def _projection(h_tok, Wp, bp, wt):
    return pl.pallas_call(
        _proj_block,
        grid=(N // TB,),
        in_specs=[
            pl.BlockSpec((TB, F), lambda i: (i, 0)),
            pl.BlockSpec((D, F), lambda i: (0, 0)),
            pl.BlockSpec((1, D), lambda i: (0, 0)),
            pl.BlockSpec((TB, 1), lambda i: (i, 0)),
        ],
        out_specs=pl.BlockSpec((TB, D), lambda i: (i, 0)),
        out_shape=jax.ShapeDtypeStruct((N, D), jnp.float32),
    )(h_tok, Wp, bp.reshape(1, D), wt)


def _gather_body(slots_hbm, y_hbm, out_hbm, sv, ybuf, sem):
    wid = lax.axis_index("c") * NS + lax.axis_index("s")
    base_tok = wid * CHUNK
    pltpu.sync_copy(slots_hbm.at[pl.ds(base_tok, CHUNK)], sv)
    pltpu.async_copy(y_hbm.at[sv], ybuf, sem).wait()
    pltpu.sync_copy(ybuf, out_hbm.at[pl.ds(base_tok, CHUNK)])


def _gather(slots, y_disp):
    mesh = plsc.VectorSubcoreMesh(core_axis_name="c", subcore_axis_name="s",
                                  num_cores=NC, num_subcores=NS)
    f = pl.kernel(
        _gather_body,
        out_type=jax.ShapeDtypeStruct((N, D), jnp.float32),
        mesh=mesh,
        scratch_types=[
            pltpu.VMEM((CHUNK,), jnp.int32),
            pltpu.VMEM((CHUNK, D), jnp.float32),
            pltpu.SemaphoreType.DMA,
        ],
        compiler_params=pltpu.CompilerParams(needs_layout_passes=False),
    )
    return f(slots, y_disp)


def kernel(x, Wg, bg, W_experts, b_experts, Wp, bp, noise):
    orig_shape = x.shape
    x_flat = x.reshape(N, D)
    wt, eid, counts = _gating(x_flat, Wg, bg, noise)
    x_disp, slots, be_arr = _dispatch(
        eid.reshape(N), counts.reshape(NW * 16), x_flat)
    h_disp = _expert_mm(be_arr, x_disp, W_experts, b_experts)
    h_tok = _gather(slots, h_disp)
    out = _projection(h_tok, Wp, bp, wt)
    return out.reshape(orig_shape)
